# baseline (speedup 1.0000x reference)
import jax
import jax.numpy as jnp
from jax.experimental import pallas as pl
from jax.experimental.pallas import tpu as pltpu

_BLK = 256


def _colsum_exp_body(a_ref, p_ref):
    e = jnp.exp(a_ref[...])
    p_ref[0] = jnp.sum(e, axis=0, keepdims=True)


def _mid_body(a_ref, u_ref, r_ref):
    e = jnp.exp(a_ref[...])
    s = jnp.sum(e * u_ref[...], axis=1, keepdims=True)
    r_ref[0] = jnp.sum(e * (1.0 / s), axis=0, keepdims=True)


def _final_body(a_ref, u_ref, o_ref):
    w = jnp.exp(a_ref[...]) * u_ref[...]
    s = jnp.sum(w, axis=1, keepdims=True)
    o_ref[...] = w * (1.0 / s)


def kernel(logits):
    j_dim, i_dim = logits.shape
    grid = (j_dim // _BLK,)
    g = grid[0]

    a_spec = pl.BlockSpec((_BLK, i_dim), lambda b: (b, 0))
    u_spec = pl.BlockSpec((1, i_dim), lambda b: (0, 0))
    part_shape = jax.ShapeDtypeStruct((g, 1, i_dim), jnp.float32)
    part_spec = pl.BlockSpec((1, 1, i_dim), lambda b: (b, 0, 0))
    params = pltpu.CompilerParams(
        dimension_semantics=("parallel",),
        vmem_limit_bytes=100 * 1024 * 1024,
    )

    k_f = float(j_dim)
    b_f = float(i_dim)

    p1 = pl.pallas_call(
        _colsum_exp_body, grid=grid, in_specs=[a_spec],
        out_specs=part_spec, out_shape=part_shape,
        compiler_params=params, name="sk_colsum_exp",
    )(logits)
    u1 = 1.0 / (k_f * jnp.sum(p1, axis=0))

    r2 = pl.pallas_call(
        _mid_body, grid=grid, in_specs=[a_spec, u_spec],
        out_specs=part_spec, out_shape=part_shape,
        compiler_params=params, name="sk_mid1",
    )(logits, u1)
    u2 = b_f / (k_f * jnp.sum(r2, axis=0))

    r3 = pl.pallas_call(
        _mid_body, grid=grid, in_specs=[a_spec, u_spec],
        out_specs=part_spec, out_shape=part_shape,
        compiler_params=params, name="sk_mid2",
    )(logits, u2)
    u3 = b_f / (k_f * jnp.sum(r3, axis=0))

    out = pl.pallas_call(
        _final_body, grid=grid, in_specs=[a_spec, u_spec],
        out_specs=pl.BlockSpec((_BLK, i_dim), lambda b: (b, 0)),
        out_shape=jax.ShapeDtypeStruct((j_dim, i_dim), jnp.float32),
        compiler_params=params, name="sk_final",
    )(logits, u3)
    return out


# speedup vs baseline: 2.1534x; 2.1534x over previous
import jax
import jax.numpy as jnp
from jax.experimental import pallas as pl
from jax.experimental.pallas import tpu as pltpu

_KB = 512
_NCORES = 2


def _lane_fold(x, width=128):
    t = x[:, :width]
    for c in range(1, x.shape[1] // width):
        t = t + x[:, c * width:(c + 1) * width]
    return t


def _acc_update(acc_ref, part, b):
    @pl.when(b == 0)
    def _():
        acc_ref[0] = jnp.zeros(acc_ref.shape[1:], acc_ref.dtype)
    acc_ref[0] += part


def _pass1_body(p_ref, acc_ref):
    e = jnp.exp(p_ref[...])
    part = jnp.sum(_lane_fold(e), axis=1, keepdims=True)
    _acc_update(acc_ref, part, pl.program_id(1))


def _mid_body(p_ref, u_ref, acc_ref):
    e = jnp.exp(p_ref[...])
    s = jnp.sum(e * u_ref[...], axis=0, keepdims=True)
    r = e * (1.0 / s)
    part = jnp.sum(_lane_fold(r), axis=1, keepdims=True)
    _acc_update(acc_ref, part, pl.program_id(1))


def _final_body(p_ref, u_ref, o_ref):
    w = jnp.exp(p_ref[...]) * u_ref[...]
    s = jnp.sum(w, axis=0, keepdims=True)
    o_ref[...] = w * (1.0 / s)


def kernel(logits):
    j_dim, i_dim = logits.shape
    p = logits.T
    g2 = j_dim // _KB // _NCORES
    grid = (_NCORES, g2)

    p_spec = pl.BlockSpec((i_dim, _KB), lambda c, b: (0, c * g2 + b))
    u_spec = pl.BlockSpec((i_dim, 1), lambda c, b: (0, 0))
    acc_shape = jax.ShapeDtypeStruct((_NCORES, i_dim, 1), jnp.float32)
    acc_spec = pl.BlockSpec((1, i_dim, 1), lambda c, b: (c, 0, 0))
    params = pltpu.CompilerParams(
        dimension_semantics=("parallel", "arbitrary"),
        vmem_limit_bytes=100 * 1024 * 1024,
    )
    k_f = float(j_dim)
    b_f = float(i_dim)

    p1 = pl.pallas_call(
        _pass1_body, grid=grid, in_specs=[p_spec],
        out_specs=acc_spec, out_shape=acc_shape,
        compiler_params=params, name="sk_pass1",
    )(p)
    u1 = 1.0 / (k_f * jnp.sum(p1, axis=0))

    r2 = pl.pallas_call(
        _mid_body, grid=grid, in_specs=[p_spec, u_spec],
        out_specs=acc_spec, out_shape=acc_shape,
        compiler_params=params, name="sk_mid1",
    )(p, u1)
    u2 = b_f / (k_f * jnp.sum(r2, axis=0))

    r3 = pl.pallas_call(
        _mid_body, grid=grid, in_specs=[p_spec, u_spec],
        out_specs=acc_spec, out_shape=acc_shape,
        compiler_params=params, name="sk_mid2",
    )(p, u2)
    u3 = b_f / (k_f * jnp.sum(r3, axis=0))

    out_m = pl.pallas_call(
        _final_body, grid=grid, in_specs=[p_spec, u_spec],
        out_specs=pl.BlockSpec((i_dim, _KB), lambda c, b: (0, c * g2 + b)),
        out_shape=jax.ShapeDtypeStruct((i_dim, j_dim), jnp.float32),
        compiler_params=params, name="sk_final",
    )(p, u3)
    return out_m.T


# speedup vs baseline: 2.3134x; 1.0743x over previous
import jax
import jax.numpy as jnp
from jax.experimental import pallas as pl
from jax.experimental.pallas import tpu as pltpu

_KB = 512
_NCORES = 2


def _lane_fold(x, width=128):
    t = x[:, :width]
    for c in range(1, x.shape[1] // width):
        t = t + x[:, c * width:(c + 1) * width]
    return t


def _acc_update(acc_ref, part, b):
    @pl.when(b == 0)
    def _():
        acc_ref[0] = jnp.zeros(acc_ref.shape[1:], acc_ref.dtype)
    acc_ref[0] += part


def _pass1_body(p_ref, eb_ref, acc_ref):
    e = jnp.exp(p_ref[...])
    eb_ref[...] = e.astype(jnp.bfloat16)
    part = jnp.sum(_lane_fold(e), axis=1, keepdims=True)
    _acc_update(acc_ref, part, pl.program_id(1))


def _mid_body(eb_ref, u_ref, acc_ref):
    e = eb_ref[...].astype(jnp.float32)
    s = jnp.sum(e * u_ref[...], axis=0, keepdims=True)
    r = e * (1.0 / s)
    part = jnp.sum(_lane_fold(r), axis=1, keepdims=True)
    _acc_update(acc_ref, part, pl.program_id(1))


def _final_body(eb_ref, u_ref, o_ref):
    w = eb_ref[...].astype(jnp.float32) * u_ref[...]
    s = jnp.sum(w, axis=0, keepdims=True)
    o_ref[...] = w * (1.0 / s)


def kernel(logits):
    j_dim, i_dim = logits.shape
    p = logits.T
    g2 = j_dim // _KB // _NCORES
    grid = (_NCORES, g2)

    p_spec = pl.BlockSpec((i_dim, _KB), lambda c, b: (0, c * g2 + b))
    u_spec = pl.BlockSpec((i_dim, 1), lambda c, b: (0, 0))
    acc_shape = jax.ShapeDtypeStruct((_NCORES, i_dim, 1), jnp.float32)
    acc_spec = pl.BlockSpec((1, i_dim, 1), lambda c, b: (c, 0, 0))
    params = pltpu.CompilerParams(
        dimension_semantics=("parallel", "arbitrary"),
        vmem_limit_bytes=100 * 1024 * 1024,
    )
    k_f = float(j_dim)
    b_f = float(i_dim)

    eb, p1 = pl.pallas_call(
        _pass1_body, grid=grid, in_specs=[p_spec],
        out_specs=(p_spec, acc_spec),
        out_shape=(jax.ShapeDtypeStruct((i_dim, j_dim), jnp.bfloat16),
                   acc_shape),
        compiler_params=params, name="sk_pass1",
    )(p)
    u1 = 1.0 / (k_f * jnp.sum(p1, axis=0))

    r2 = pl.pallas_call(
        _mid_body, grid=grid, in_specs=[p_spec, u_spec],
        out_specs=acc_spec, out_shape=acc_shape,
        compiler_params=params, name="sk_mid1",
    )(eb, u1)
    u2 = b_f / (k_f * jnp.sum(r2, axis=0))

    r3 = pl.pallas_call(
        _mid_body, grid=grid, in_specs=[p_spec, u_spec],
        out_specs=acc_spec, out_shape=acc_shape,
        compiler_params=params, name="sk_mid2",
    )(eb, u2)
    u3 = b_f / (k_f * jnp.sum(r3, axis=0))

    out_m = pl.pallas_call(
        _final_body, grid=grid, in_specs=[p_spec, u_spec],
        out_specs=pl.BlockSpec((i_dim, _KB), lambda c, b: (0, c * g2 + b)),
        out_shape=jax.ShapeDtypeStruct((i_dim, j_dim), jnp.float32),
        compiler_params=params, name="sk_final",
    )(eb, u3)
    return out_m.T


# speedup vs baseline: 2.4379x; 1.0538x over previous
import jax
import jax.numpy as jnp
from jax.experimental import pallas as pl
from jax.experimental.pallas import tpu as pltpu

_KB = 512
_KB_MID = 1024
_NCORES = 2


def _lane_fold(x, width=128):
    t = x[:, :width]
    for c in range(1, x.shape[1] // width):
        t = t + x[:, c * width:(c + 1) * width]
    return t


def _acc_update(acc_ref, part, b):
    @pl.when(b == 0)
    def _():
        acc_ref[0] = jnp.zeros(acc_ref.shape[1:], acc_ref.dtype)
    acc_ref[0] += part


def _pass1_body(p_ref, eb_ref, acc_ref):
    e = jnp.exp(p_ref[...])
    eb_ref[...] = e.astype(jnp.bfloat16)
    part = jnp.sum(_lane_fold(e), axis=1, keepdims=True)
    _acc_update(acc_ref, part, pl.program_id(1))


def _mid_body(eb_ref, u_ref, acc_ref):
    e = eb_ref[...].astype(jnp.float32)
    s = jnp.sum(e * u_ref[...], axis=0, keepdims=True)
    r = e * (1.0 / s)
    part = jnp.sum(_lane_fold(r), axis=1, keepdims=True)
    _acc_update(acc_ref, part, pl.program_id(1))


def _final_body(eb_ref, u_ref, o_ref):
    w = eb_ref[...].astype(jnp.float32) * u_ref[...]
    s = jnp.sum(w, axis=0, keepdims=True)
    o_ref[...] = w * (1.0 / s)


def kernel(logits):
    j_dim, i_dim = logits.shape
    p = logits.T
    g2 = j_dim // _KB // _NCORES
    grid = (_NCORES, g2)

    g2m = j_dim // _KB_MID // _NCORES
    grid_m = (_NCORES, g2m)

    p_spec = pl.BlockSpec((i_dim, _KB), lambda c, b: (0, c * g2 + b))
    pm_spec = pl.BlockSpec((i_dim, _KB_MID), lambda c, b: (0, c * g2m + b))
    u_spec = pl.BlockSpec((i_dim, 1), lambda c, b: (0, 0))
    acc_shape = jax.ShapeDtypeStruct((_NCORES, i_dim, 1), jnp.float32)
    acc_spec = pl.BlockSpec((1, i_dim, 1), lambda c, b: (c, 0, 0))
    params = pltpu.CompilerParams(
        dimension_semantics=("parallel", "arbitrary"),
        vmem_limit_bytes=100 * 1024 * 1024,
    )
    k_f = float(j_dim)
    b_f = float(i_dim)

    eb, p1 = pl.pallas_call(
        _pass1_body, grid=grid, in_specs=[p_spec],
        out_specs=(p_spec, acc_spec),
        out_shape=(jax.ShapeDtypeStruct((i_dim, j_dim), jnp.bfloat16),
                   acc_shape),
        compiler_params=params, name="sk_pass1",
    )(p)
    u1 = 1.0 / (k_f * jnp.sum(p1, axis=0))

    r2 = pl.pallas_call(
        _mid_body, grid=grid_m, in_specs=[pm_spec, u_spec],
        out_specs=acc_spec, out_shape=acc_shape,
        compiler_params=params, name="sk_mid1",
    )(eb, u1)
    u2 = b_f / (k_f * jnp.sum(r2, axis=0))

    r3 = pl.pallas_call(
        _mid_body, grid=grid_m, in_specs=[pm_spec, u_spec],
        out_specs=acc_spec, out_shape=acc_shape,
        compiler_params=params, name="sk_mid2",
    )(eb, u2)
    u3 = b_f / (k_f * jnp.sum(r3, axis=0))

    out_m = pl.pallas_call(
        _final_body, grid=grid_m, in_specs=[pm_spec, u_spec],
        out_specs=pl.BlockSpec((i_dim, _KB_MID), lambda c, b: (0, c * g2m + b)),
        out_shape=jax.ShapeDtypeStruct((i_dim, j_dim), jnp.float32),
        compiler_params=params, name="sk_final",
    )(eb, u3)
    return out_m.T


# speedup vs baseline: 2.4434x; 1.0022x over previous
import jax
import jax.numpy as jnp
from jax.experimental import pallas as pl
from jax.experimental.pallas import tpu as pltpu

_KB = 512
_KB_MID = 1024
_NCORES = 2


_CHUNK = 256


def _acc_update(acc_ref, part, b):
    @pl.when(b == 0)
    def _():
        acc_ref[0] = jnp.zeros(acc_ref.shape[1:], acc_ref.dtype)
    acc_ref[0] += part


def _pass1_body(p_ref, eb_ref, acc_ref):
    fold = None
    for c in range(p_ref.shape[1] // _CHUNK):
        sl = slice(c * _CHUNK, (c + 1) * _CHUNK)
        ec = jnp.exp(p_ref[:, sl])
        eb_ref[:, sl] = ec.astype(jnp.bfloat16)
        fold = ec if fold is None else fold + ec
    part = jnp.sum(fold, axis=1, keepdims=True)
    _acc_update(acc_ref, part, pl.program_id(1))


def _mid_body(eb_ref, u_ref, acc_ref):
    u = u_ref[...]
    fold = None
    for c in range(eb_ref.shape[1] // _CHUNK):
        sl = slice(c * _CHUNK, (c + 1) * _CHUNK)
        ec = eb_ref[:, sl].astype(jnp.float32)
        sc = jnp.sum(ec * u, axis=0, keepdims=True)
        rc = ec * (1.0 / sc)
        fold = rc if fold is None else fold + rc
    part = jnp.sum(fold, axis=1, keepdims=True)
    _acc_update(acc_ref, part, pl.program_id(1))


def _final_body(eb_ref, u_ref, o_ref):
    u = u_ref[...]
    for c in range(eb_ref.shape[1] // _CHUNK):
        sl = slice(c * _CHUNK, (c + 1) * _CHUNK)
        wc = eb_ref[:, sl].astype(jnp.float32) * u
        sc = jnp.sum(wc, axis=0, keepdims=True)
        o_ref[:, sl] = wc * (1.0 / sc)


def kernel(logits):
    j_dim, i_dim = logits.shape
    p = logits.T
    g2 = j_dim // _KB // _NCORES
    grid = (_NCORES, g2)

    g2m = j_dim // _KB_MID // _NCORES
    grid_m = (_NCORES, g2m)

    p_spec = pl.BlockSpec((i_dim, _KB), lambda c, b: (0, c * g2 + b))
    pm_spec = pl.BlockSpec((i_dim, _KB_MID), lambda c, b: (0, c * g2m + b))
    u_spec = pl.BlockSpec((i_dim, 1), lambda c, b: (0, 0))
    acc_shape = jax.ShapeDtypeStruct((_NCORES, i_dim, 1), jnp.float32)
    acc_spec = pl.BlockSpec((1, i_dim, 1), lambda c, b: (c, 0, 0))
    params = pltpu.CompilerParams(
        dimension_semantics=("parallel", "arbitrary"),
        vmem_limit_bytes=56 * 1024 * 1024,
    )
    k_f = float(j_dim)
    b_f = float(i_dim)

    eb, p1 = pl.pallas_call(
        _pass1_body, grid=grid, in_specs=[p_spec],
        out_specs=(p_spec, acc_spec),
        out_shape=(jax.ShapeDtypeStruct((i_dim, j_dim), jnp.bfloat16),
                   acc_shape),
        compiler_params=params, name="sk_pass1",
    )(p)
    u1 = 1.0 / (k_f * jnp.sum(p1, axis=0))

    r2 = pl.pallas_call(
        _mid_body, grid=grid_m, in_specs=[pm_spec, u_spec],
        out_specs=acc_spec, out_shape=acc_shape,
        compiler_params=params, name="sk_mid1",
    )(eb, u1)
    u2 = b_f / (k_f * jnp.sum(r2, axis=0))

    r3 = pl.pallas_call(
        _mid_body, grid=grid_m, in_specs=[pm_spec, u_spec],
        out_specs=acc_spec, out_shape=acc_shape,
        compiler_params=params, name="sk_mid2",
    )(eb, u2)
    u3 = b_f / (k_f * jnp.sum(r3, axis=0))

    out_m = pl.pallas_call(
        _final_body, grid=grid_m, in_specs=[pm_spec, u_spec],
        out_specs=pl.BlockSpec((i_dim, _KB_MID), lambda c, b: (0, c * g2m + b)),
        out_shape=jax.ShapeDtypeStruct((i_dim, j_dim), jnp.float32),
        compiler_params=params, name="sk_final",
    )(eb, u3)
    return out_m.T
